# Initial kernel scaffold; baseline (speedup 1.0000x reference)
import jax
import jax.numpy as jnp
from jax.experimental import pallas as pl


def kernel(x, edge_index, batch, W1, b1, Wg, att_src, att_dst, bg, W2, b2):
    raise NotImplementedError("write your pallas kernel here")


# speedup vs baseline: 23.8963x; 23.8963x over previous
import jax
import jax.numpy as jnp
from jax import lax
from jax.experimental import pallas as pl
from jax.experimental.pallas import tpu as pltpu
from jax.experimental.pallas import tpu_sc as plsc

N = 10000
D = 128
H = 128
OUT = 128
G = 64
NP = 10240
NPT = NP // 16
EE = 320000 + N
NCHUNK = 82
EROWS = 32 * NCHUNK
EEP = EROWS * 128
PAD = EEP - EE
RB = 10
_HI = jax.lax.Precision.HIGHEST


def _dot(a, b):
  return jax.lax.dot(a, b, precision=_HI, preferred_element_type=jnp.float32)


def _tc1_body(x_ref, w1_ref, b1_ref, wg_ref, asrc_ref, adst_ref,
              h0_ref, hw_ref, asv_ref, adv_ref, mx_ref):
  h0 = _dot(x_ref[...], w1_ref[...]) + b1_ref[...]
  hw = _dot(h0, wg_ref[...])
  asv = _dot(hw, asrc_ref[...])
  adv = _dot(hw, adst_ref[...])
  h0_ref[...] = h0
  hw_ref[...] = hw
  asv_ref[...] = asv
  adv_ref[...] = adv
  ma = jnp.broadcast_to(jnp.max(asv), (1, 1, 64))
  mb = jnp.broadcast_to(jnp.max(adv), (1, 1, 64))
  mx_ref[...] = jnp.concatenate([ma, mb], axis=2)


_tc1 = pl.pallas_call(
    _tc1_body,
    grid=(RB,),
    in_specs=[
        pl.BlockSpec((1024, D), lambda i: (i, 0)),
        pl.BlockSpec((D, H), lambda i: (0, 0)),
        pl.BlockSpec((1, H), lambda i: (0, 0)),
        pl.BlockSpec((H, H), lambda i: (0, 0)),
        pl.BlockSpec((H, 1), lambda i: (0, 0)),
        pl.BlockSpec((H, 1), lambda i: (0, 0)),
    ],
    out_specs=[
        pl.BlockSpec((1024, H), lambda i: (i, 0)),
        pl.BlockSpec((1024, H), lambda i: (i, 0)),
        pl.BlockSpec((1024, 1), lambda i: (i, 0)),
        pl.BlockSpec((1024, 1), lambda i: (i, 0)),
        pl.BlockSpec((1, 1, 128), lambda i: (i, 0, 0)),
    ],
    out_shape=[
        jax.ShapeDtypeStruct((NP, H), jnp.float32),
        jax.ShapeDtypeStruct((NP, H), jnp.float32),
        jax.ShapeDtypeStruct((NP, 1), jnp.float32),
        jax.ShapeDtypeStruct((NP, 1), jnp.float32),
        jax.ShapeDtypeStruct((RB, 1, 128), jnp.float32),
    ],
)


def _tc2_body(o0_ref, o1_ref, sb0_ref, sb1_ref, bgl_ref, hin_ref, wg_ref,
              asrc_ref, adst_ref, h1_ref, hw_ref, asv_ref, adv_ref, mx_ref):
  denom = sb0_ref[...] + sb1_ref[...] + 1e-16
  agg = (o0_ref[...] + o1_ref[...]) / denom + bgl_ref[...]
  h1 = jnp.maximum(agg, 0.0) + hin_ref[...]
  hw = _dot(h1, wg_ref[...])
  asv = _dot(hw, asrc_ref[...])
  adv = _dot(hw, adst_ref[...])
  h1_ref[...] = h1
  hw_ref[...] = hw
  asv_ref[...] = asv
  adv_ref[...] = adv
  ma = jnp.broadcast_to(jnp.max(asv), (1, 1, 64))
  mb = jnp.broadcast_to(jnp.max(adv), (1, 1, 64))
  mx_ref[...] = jnp.concatenate([ma, mb], axis=2)


_tc2 = pl.pallas_call(
    _tc2_body,
    grid=(RB,),
    in_specs=[
        pl.BlockSpec((1024, H), lambda i: (i, 0)),
        pl.BlockSpec((1024, H), lambda i: (i, 0)),
        pl.BlockSpec((1024, H), lambda i: (i, 0)),
        pl.BlockSpec((1024, H), lambda i: (i, 0)),
        pl.BlockSpec((1, H), lambda i: (0, 0)),
        pl.BlockSpec((1024, H), lambda i: (i, 0)),
        pl.BlockSpec((H, H), lambda i: (0, 0)),
        pl.BlockSpec((H, 1), lambda i: (0, 0)),
        pl.BlockSpec((H, 1), lambda i: (0, 0)),
    ],
    out_specs=[
        pl.BlockSpec((1024, H), lambda i: (i, 0)),
        pl.BlockSpec((1024, H), lambda i: (i, 0)),
        pl.BlockSpec((1024, 1), lambda i: (i, 0)),
        pl.BlockSpec((1024, 1), lambda i: (i, 0)),
        pl.BlockSpec((1, 1, 128), lambda i: (i, 0, 0)),
    ],
    out_shape=[
        jax.ShapeDtypeStruct((NP, H), jnp.float32),
        jax.ShapeDtypeStruct((NP, H), jnp.float32),
        jax.ShapeDtypeStruct((NP, 1), jnp.float32),
        jax.ShapeDtypeStruct((NP, 1), jnp.float32),
        jax.ShapeDtypeStruct((RB, 1, 128), jnp.float32),
    ],
)


def _tc3_body(o0_ref, o1_ref, sb0_ref, sb1_ref, bgl_ref, hin_ref, batch_ref,
              w2_ref, b2_ref, pooled_ref, out_ref):
  i = pl.program_id(0)
  denom = sb0_ref[...] + sb1_ref[...] + 1e-16
  agg = (o0_ref[...] + o1_ref[...]) / denom + bgl_ref[...]
  h2 = jnp.maximum(agg, 0.0) + hin_ref[...]
  bt = batch_ref[...].reshape(1, 1024)
  gi = jax.lax.broadcasted_iota(jnp.int32, (G, 1024), 0)
  onehot = (gi == bt).astype(jnp.float32)
  part = _dot(onehot, h2)

  @pl.when(i == 0)
  def _():
    pooled_ref[...] = part

  @pl.when(i > 0)
  def _():
    pooled_ref[...] = pooled_ref[...] + part

  @pl.when(i == RB - 1)
  def _():
    out_ref[...] = _dot(pooled_ref[...], w2_ref[...]) + b2_ref[...]


_tc3 = pl.pallas_call(
    _tc3_body,
    grid=(RB,),
    in_specs=[
        pl.BlockSpec((1024, H), lambda i: (i, 0)),
        pl.BlockSpec((1024, H), lambda i: (i, 0)),
        pl.BlockSpec((1024, H), lambda i: (i, 0)),
        pl.BlockSpec((1024, H), lambda i: (i, 0)),
        pl.BlockSpec((1, H), lambda i: (0, 0)),
        pl.BlockSpec((1024, H), lambda i: (i, 0)),
        pl.BlockSpec((1024,), lambda i: (i,)),
        pl.BlockSpec((H, OUT), lambda i: (0, 0)),
        pl.BlockSpec((1, OUT), lambda i: (0, 0)),
    ],
    out_specs=[
        pl.BlockSpec((G, H), lambda i: (0, 0)),
        pl.BlockSpec((G, OUT), lambda i: (0, 0)),
    ],
    out_shape=[
        jax.ShapeDtypeStruct((G, H), jnp.float32),
        jax.ShapeDtypeStruct((G, OUT), jnp.float32),
    ],
)


def _sc_layer_body(src3_h, dst3_h, asv_h, adv_h, mvec_h, hw_h,
                   o_out, sb_out,
                   idxs, idxd, asg, adg, w_t, rows, sv_t, m_t,
                   s_sh, o_sh, sem_r, sem_s):
  cid = lax.axis_index("c")
  sid = lax.axis_index("s")
  z16 = jnp.zeros((16,), jnp.float32)

  @pl.loop(0, 128)
  def _zrows(j):
    for k in range(8):
      rows[j, pl.ds(16 * k, 16)] = z16

  for j in range(NPT // 16):
    sv_t[pl.ds(16 * j, 16)] = z16
  for b in range(NPT // 128):
    pltpu.sync_copy(rows, o_sh.at[pl.ds(sid * NPT + b * 128, 128)])
  pltpu.sync_copy(sv_t, s_sh.at[pl.ds(sid * NPT, NPT)])
  pltpu.sync_copy(mvec_h, m_t)
  plsc.subcore_barrier()

  m16 = m_t[...]
  tb = (cid * 16 + sid) * NCHUNK

  @pl.loop(0, NCHUNK)
  def _chunk(c):
    ci = tb + c
    pltpu.sync_copy(src3_h.at[ci], idxs)
    pltpu.sync_copy(dst3_h.at[ci], idxd)
    cp_rows = pltpu.async_copy(hw_h.at[idxs.at[0]], rows, sem_r)
    cp_as = pltpu.async_copy(asv_h.at[idxs.at[0]], asg, sem_s)
    cp_ad = pltpu.async_copy(adv_h.at[idxd.at[0]], adg, sem_s)
    cp_as.wait()
    cp_ad.wait()
    for i in range(8):
      sl = pl.ds(16 * i, 16)
      t = asg[sl] + adg[sl]
      e = jnp.where(t > 0, t, 0.2 * t)
      w_t[sl] = jnp.exp(e - m16)
    pltpu.sync_copy(w_t, s_sh.at[idxd.at[0]], add=True)
    cp_rows.wait()

    @pl.loop(0, 128)
    def _scale(j):
      wb = plsc.load_gather(w_t, [jnp.full((16,), j, jnp.int32)])
      for k in range(8):
        sl = pl.ds(16 * k, 16)
        rows[j, sl] = rows[j, sl] * wb

    pltpu.sync_copy(rows, o_sh.at[idxd.at[0]], add=True)

  plsc.subcore_barrier()

  pltpu.sync_copy(o_sh.at[pl.ds(sid * NPT, NPT)],
                  o_out.at[cid, pl.ds(sid * NPT, NPT)])
  pltpu.sync_copy(s_sh.at[pl.ds(sid * NPT, NPT)], sv_t)

  for b in range(NPT // 128):
    @pl.loop(0, 128)
    def _bcast(j):
      wb = plsc.load_gather(sv_t, [jnp.full((16,), b * 128 + j, jnp.int32)])
      for k in range(8):
        rows[j, pl.ds(16 * k, 16)] = wb

    pltpu.sync_copy(rows, sb_out.at[cid, pl.ds(sid * NPT + b * 128, 128)])


_sc_layer = pl.kernel(
    _sc_layer_body,
    out_type=(
        jax.ShapeDtypeStruct((2, NP, H), jnp.float32),
        jax.ShapeDtypeStruct((2, NP, H), jnp.float32),
    ),
    mesh=plsc.VectorSubcoreMesh(core_axis_name="c", subcore_axis_name="s",
                                num_cores=2, num_subcores=16),
    compiler_params=pltpu.CompilerParams(needs_layout_passes=False),
    scratch_types=[
        pltpu.VMEM((1, 128), jnp.int32),
        pltpu.VMEM((1, 128), jnp.int32),
        pltpu.VMEM((128,), jnp.float32),
        pltpu.VMEM((128,), jnp.float32),
        pltpu.VMEM((128,), jnp.float32),
        pltpu.VMEM((128, H), jnp.float32),
        pltpu.VMEM((NPT,), jnp.float32),
        pltpu.VMEM((16,), jnp.float32),
        pltpu.VMEM_SHARED((NP,), jnp.float32),
        pltpu.VMEM_SHARED((NP, H), jnp.float32),
        pltpu.SemaphoreType.DMA,
        pltpu.SemaphoreType.DMA,
    ],
)


def _shift_const(mx):
  tmax = jnp.max(mx[:, 0, :64]) + jnp.max(mx[:, 0, 64:])
  return jnp.where(tmax > 0, tmax, 0.2 * tmax)


def kernel(x, edge_index, batch, W1, b1, Wg, att_src, att_dst, bg, W2, b2):
  f32 = jnp.float32
  xp = jnp.zeros((NP, D), f32).at[:N].set(x)
  loops = jnp.arange(N, dtype=jnp.int32)
  pad_src = (jnp.arange(PAD, dtype=jnp.int32) * 37) % N
  pad_dst = N + (jnp.arange(PAD, dtype=jnp.int32) % (NP - N))
  src3 = jnp.concatenate(
      [edge_index[0].astype(jnp.int32), loops, pad_src]).reshape(EROWS, 1, 128)
  dst3 = jnp.concatenate(
      [edge_index[1].astype(jnp.int32), loops, pad_dst]).reshape(EROWS, 1, 128)
  batchp = jnp.full((NP,), G, jnp.int32).at[:N].set(batch.astype(jnp.int32))

  h0, hw, asv, adv, mx = _tc1(xp, W1, b1.reshape(1, H), Wg[0],
                              att_src[0].reshape(H, 1),
                              att_dst[0].reshape(H, 1))
  h_in = h0
  for l in range(2):
    mvec = jnp.full((16,), _shift_const(mx), f32)
    o2, sb2 = _sc_layer(src3, dst3, asv.reshape(NP), adv.reshape(NP),
                        mvec, hw)
    if l == 0:
      h_in, hw, asv, adv, mx = _tc2(o2[0], o2[1], sb2[0], sb2[1],
                                    bg[0].reshape(1, H), h_in, Wg[1],
                                    att_src[1].reshape(H, 1),
                                    att_dst[1].reshape(H, 1))
    else:
      _, out = _tc3(o2[0], o2[1], sb2[0], sb2[1], bg[1].reshape(1, H), h_in,
                    batchp, W2, b2.reshape(1, OUT))
  return out


# speedup vs baseline: 35.8456x; 1.5001x over previous
import jax
import jax.numpy as jnp
from jax import lax
from jax.experimental import pallas as pl
from jax.experimental.pallas import tpu as pltpu
from jax.experimental.pallas import tpu_sc as plsc

N = 10000
D = 128
H = 128
OUT = 128
G = 64
NP = 10240
NPT = NP // 16
EE = 320000 + N
NCHUNK = 82
EROWS = 32 * NCHUNK
EEP = EROWS * 128
PAD = EEP - EE
RB = 10
_HI = jax.lax.Precision.HIGHEST


def _dot(a, b):
  return jax.lax.dot(a, b, precision=_HI, preferred_element_type=jnp.float32)


def _tc1_body(x_ref, w1_ref, b1_ref, wg_ref, asrc_ref, adst_ref,
              h0_ref, hw_ref, asv_ref, adv_ref, mx_ref):
  h0 = _dot(x_ref[...], w1_ref[...]) + b1_ref[...]
  hw = _dot(h0, wg_ref[...])
  asv = _dot(hw, asrc_ref[...])
  adv = _dot(hw, adst_ref[...])
  h0_ref[...] = h0
  hw_ref[...] = hw
  asv_ref[...] = asv
  adv_ref[...] = adv
  ma = jnp.broadcast_to(jnp.max(asv), (1, 1, 64))
  mb = jnp.broadcast_to(jnp.max(adv), (1, 1, 64))
  mx_ref[...] = jnp.concatenate([ma, mb], axis=2)


_tc1 = pl.pallas_call(
    _tc1_body,
    grid=(RB,),
    in_specs=[
        pl.BlockSpec((1024, D), lambda i: (i, 0)),
        pl.BlockSpec((D, H), lambda i: (0, 0)),
        pl.BlockSpec((1, H), lambda i: (0, 0)),
        pl.BlockSpec((H, H), lambda i: (0, 0)),
        pl.BlockSpec((H, 1), lambda i: (0, 0)),
        pl.BlockSpec((H, 1), lambda i: (0, 0)),
    ],
    out_specs=[
        pl.BlockSpec((1024, H), lambda i: (i, 0)),
        pl.BlockSpec((1024, H), lambda i: (i, 0)),
        pl.BlockSpec((1024, 1), lambda i: (i, 0)),
        pl.BlockSpec((1024, 1), lambda i: (i, 0)),
        pl.BlockSpec((1, 1, 128), lambda i: (i, 0, 0)),
    ],
    out_shape=[
        jax.ShapeDtypeStruct((NP, H), jnp.float32),
        jax.ShapeDtypeStruct((NP, H), jnp.float32),
        jax.ShapeDtypeStruct((NP, 1), jnp.float32),
        jax.ShapeDtypeStruct((NP, 1), jnp.float32),
        jax.ShapeDtypeStruct((RB, 1, 128), jnp.float32),
    ],
)


def _tc2_body(o0_ref, o1_ref, sb0_ref, sb1_ref, bgl_ref, hin_ref, wg_ref,
              asrc_ref, adst_ref, h1_ref, hw_ref, asv_ref, adv_ref, mx_ref):
  denom = sb0_ref[...] + sb1_ref[...] + 1e-16
  agg = (o0_ref[...] + o1_ref[...]) / denom + bgl_ref[...]
  h1 = jnp.maximum(agg, 0.0) + hin_ref[...]
  hw = _dot(h1, wg_ref[...])
  asv = _dot(hw, asrc_ref[...])
  adv = _dot(hw, adst_ref[...])
  h1_ref[...] = h1
  hw_ref[...] = hw
  asv_ref[...] = asv
  adv_ref[...] = adv
  ma = jnp.broadcast_to(jnp.max(asv), (1, 1, 64))
  mb = jnp.broadcast_to(jnp.max(adv), (1, 1, 64))
  mx_ref[...] = jnp.concatenate([ma, mb], axis=2)


_tc2 = pl.pallas_call(
    _tc2_body,
    grid=(RB,),
    in_specs=[
        pl.BlockSpec((1024, H), lambda i: (i, 0)),
        pl.BlockSpec((1024, H), lambda i: (i, 0)),
        pl.BlockSpec((1024, H), lambda i: (i, 0)),
        pl.BlockSpec((1024, H), lambda i: (i, 0)),
        pl.BlockSpec((1, H), lambda i: (0, 0)),
        pl.BlockSpec((1024, H), lambda i: (i, 0)),
        pl.BlockSpec((H, H), lambda i: (0, 0)),
        pl.BlockSpec((H, 1), lambda i: (0, 0)),
        pl.BlockSpec((H, 1), lambda i: (0, 0)),
    ],
    out_specs=[
        pl.BlockSpec((1024, H), lambda i: (i, 0)),
        pl.BlockSpec((1024, H), lambda i: (i, 0)),
        pl.BlockSpec((1024, 1), lambda i: (i, 0)),
        pl.BlockSpec((1024, 1), lambda i: (i, 0)),
        pl.BlockSpec((1, 1, 128), lambda i: (i, 0, 0)),
    ],
    out_shape=[
        jax.ShapeDtypeStruct((NP, H), jnp.float32),
        jax.ShapeDtypeStruct((NP, H), jnp.float32),
        jax.ShapeDtypeStruct((NP, 1), jnp.float32),
        jax.ShapeDtypeStruct((NP, 1), jnp.float32),
        jax.ShapeDtypeStruct((RB, 1, 128), jnp.float32),
    ],
)


def _tc3_body(o0_ref, o1_ref, sb0_ref, sb1_ref, bgl_ref, hin_ref, batch_ref,
              w2_ref, b2_ref, pooled_ref, out_ref):
  i = pl.program_id(0)
  denom = sb0_ref[...] + sb1_ref[...] + 1e-16
  agg = (o0_ref[...] + o1_ref[...]) / denom + bgl_ref[...]
  h2 = jnp.maximum(agg, 0.0) + hin_ref[...]
  bt = batch_ref[...].reshape(1, 1024)
  gi = jax.lax.broadcasted_iota(jnp.int32, (G, 1024), 0)
  onehot = (gi == bt).astype(jnp.float32)
  part = _dot(onehot, h2)

  @pl.when(i == 0)
  def _():
    pooled_ref[...] = part

  @pl.when(i > 0)
  def _():
    pooled_ref[...] = pooled_ref[...] + part

  @pl.when(i == RB - 1)
  def _():
    out_ref[...] = _dot(pooled_ref[...], w2_ref[...]) + b2_ref[...]


_tc3 = pl.pallas_call(
    _tc3_body,
    grid=(RB,),
    in_specs=[
        pl.BlockSpec((1024, H), lambda i: (i, 0)),
        pl.BlockSpec((1024, H), lambda i: (i, 0)),
        pl.BlockSpec((1024, H), lambda i: (i, 0)),
        pl.BlockSpec((1024, H), lambda i: (i, 0)),
        pl.BlockSpec((1, H), lambda i: (0, 0)),
        pl.BlockSpec((1024, H), lambda i: (i, 0)),
        pl.BlockSpec((1024,), lambda i: (i,)),
        pl.BlockSpec((H, OUT), lambda i: (0, 0)),
        pl.BlockSpec((1, OUT), lambda i: (0, 0)),
    ],
    out_specs=[
        pl.BlockSpec((G, H), lambda i: (0, 0)),
        pl.BlockSpec((G, OUT), lambda i: (0, 0)),
    ],
    out_shape=[
        jax.ShapeDtypeStruct((G, H), jnp.float32),
        jax.ShapeDtypeStruct((G, OUT), jnp.float32),
    ],
)


def _sc_layer_body(idx3_h, asv_h, adv_h, mvec_h, hw_h,
                   o_out, sb_out,
                   idx_a, idx_b, asg_a, asg_b, adg_a, adg_b, w_a, w_b,
                   rows_a, rows_b, sv_t, m_t, s_sh, o_sh,
                   sg_a, sg_b, sr_a, sr_b, so_a, so_b):
  cid = lax.axis_index("c")
  sid = lax.axis_index("s")
  z16 = jnp.zeros((16,), jnp.float32)
  tb = (cid * 16 + sid) * NCHUNK

  def prefetch(c, idx, asg, adg, rows, sg, sr):
    pltpu.sync_copy(idx3_h.at[c], idx)
    pltpu.async_copy(asv_h.at[idx.at[0]], asg, sg)
    pltpu.async_copy(adv_h.at[idx.at[1]], adg, sg)
    pltpu.async_copy(hw_h.at[idx.at[0]], rows, sr)

  def process(idx, asg, adg, w_t, rows, sg, sr, so):
    pltpu.make_async_copy(asv_h.at[idx.at[0]], asg, sg).wait()
    pltpu.make_async_copy(adv_h.at[idx.at[1]], adg, sg).wait()
    m16 = m_t[...]
    for i in range(8):
      sl = pl.ds(16 * i, 16)
      t = asg[sl] + adg[sl]
      e = jnp.where(t > 0, t, 0.2 * t)
      w_t[sl] = jnp.exp(e - m16)
    pltpu.make_async_copy(hw_h.at[idx.at[0]], rows, sr).wait()

    @pl.loop(0, 128, unroll=4)
    def _scale(j):
      wb = plsc.load_gather(w_t, [jnp.full((16,), j, jnp.int32)])
      for k in range(8):
        sl = pl.ds(16 * k, 16)
        rows[j, sl] = rows[j, sl] * wb

    pltpu.async_copy(w_t, s_sh.at[idx.at[1]], so, add=True)
    pltpu.async_copy(rows, o_sh.at[idx.at[1]], so, add=True)

  def drain(idx, w_t, rows, so):
    pltpu.make_async_copy(w_t, s_sh.at[idx.at[1]], so).wait()
    pltpu.make_async_copy(rows, o_sh.at[idx.at[1]], so).wait()

  @pl.loop(0, 128, unroll=4)
  def _zrows(j):
    for k in range(8):
      rows_b[j, pl.ds(16 * k, 16)] = z16

  for j in range(NPT // 16):
    sv_t[pl.ds(16 * j, 16)] = z16
  pltpu.sync_copy(mvec_h, m_t)
  prefetch(tb, idx_a, asg_a, adg_a, rows_a, sg_a, sr_a)
  for b in range(NPT // 128):
    pltpu.sync_copy(rows_b, o_sh.at[pl.ds(sid * NPT + b * 128, 128)])
  pltpu.sync_copy(sv_t, s_sh.at[pl.ds(sid * NPT, NPT)])
  plsc.subcore_barrier()

  @pl.loop(0, NCHUNK // 2)
  def _pair(t):
    c0 = tb + 2 * t

    @pl.when(t > 0)
    def _():
      drain(idx_b, w_b, rows_b, so_b)

    prefetch(c0 + 1, idx_b, asg_b, adg_b, rows_b, sg_b, sr_b)
    process(idx_a, asg_a, adg_a, w_a, rows_a, sg_a, sr_a, so_a)

    @pl.when(t < NCHUNK // 2 - 1)
    def _():
      drain(idx_a, w_a, rows_a, so_a)
      prefetch(c0 + 2, idx_a, asg_a, adg_a, rows_a, sg_a, sr_a)

    process(idx_b, asg_b, adg_b, w_b, rows_b, sg_b, sr_b, so_b)

  drain(idx_a, w_a, rows_a, so_a)
  drain(idx_b, w_b, rows_b, so_b)
  plsc.subcore_barrier()

  pltpu.sync_copy(o_sh.at[pl.ds(sid * NPT, NPT)],
                  o_out.at[cid, pl.ds(sid * NPT, NPT)])
  pltpu.sync_copy(s_sh.at[pl.ds(sid * NPT, NPT)], sv_t)

  for b in range(NPT // 128):
    @pl.loop(0, 128, unroll=4)
    def _bcast(j):
      wb = plsc.load_gather(sv_t, [jnp.full((16,), b * 128 + j, jnp.int32)])
      for k in range(8):
        rows_a[j, pl.ds(16 * k, 16)] = wb

    pltpu.sync_copy(rows_a, sb_out.at[cid, pl.ds(sid * NPT + b * 128, 128)])


_sc_layer = pl.kernel(
    _sc_layer_body,
    out_type=(
        jax.ShapeDtypeStruct((2, NP, H), jnp.float32),
        jax.ShapeDtypeStruct((2, NP, H), jnp.float32),
    ),
    mesh=plsc.VectorSubcoreMesh(core_axis_name="c", subcore_axis_name="s",
                                num_cores=2, num_subcores=16),
    compiler_params=pltpu.CompilerParams(needs_layout_passes=False),
    scratch_types=[
        pltpu.VMEM((2, 128), jnp.int32),
        pltpu.VMEM((2, 128), jnp.int32),
        pltpu.VMEM((128,), jnp.float32),
        pltpu.VMEM((128,), jnp.float32),
        pltpu.VMEM((128,), jnp.float32),
        pltpu.VMEM((128,), jnp.float32),
        pltpu.VMEM((128,), jnp.float32),
        pltpu.VMEM((128,), jnp.float32),
        pltpu.VMEM((128, H), jnp.float32),
        pltpu.VMEM((128, H), jnp.float32),
        pltpu.VMEM((NPT,), jnp.float32),
        pltpu.VMEM((16,), jnp.float32),
        pltpu.VMEM_SHARED((NP,), jnp.float32),
        pltpu.VMEM_SHARED((NP, H), jnp.float32),
        pltpu.SemaphoreType.DMA,
        pltpu.SemaphoreType.DMA,
        pltpu.SemaphoreType.DMA,
        pltpu.SemaphoreType.DMA,
        pltpu.SemaphoreType.DMA,
        pltpu.SemaphoreType.DMA,
    ],
)


def _shift_const(mx):
  tmax = jnp.max(mx[:, 0, :64]) + jnp.max(mx[:, 0, 64:])
  return jnp.where(tmax > 0, tmax, 0.2 * tmax)


def kernel(x, edge_index, batch, W1, b1, Wg, att_src, att_dst, bg, W2, b2):
  f32 = jnp.float32
  xp = jnp.zeros((NP, D), f32).at[:N].set(x)
  loops = jnp.arange(N, dtype=jnp.int32)
  pad_src = (jnp.arange(PAD, dtype=jnp.int32) * 37) % N
  pad_dst = N + (jnp.arange(PAD, dtype=jnp.int32) % (NP - N))
  src_f = jnp.concatenate(
      [edge_index[0].astype(jnp.int32), loops, pad_src]).reshape(EROWS, 1, 128)
  dst_f = jnp.concatenate(
      [edge_index[1].astype(jnp.int32), loops, pad_dst]).reshape(EROWS, 1, 128)
  idx3 = jnp.concatenate([src_f, dst_f], axis=1)
  batchp = jnp.full((NP,), G, jnp.int32).at[:N].set(batch.astype(jnp.int32))

  h0, hw, asv, adv, mx = _tc1(xp, W1, b1.reshape(1, H), Wg[0],
                              att_src[0].reshape(H, 1),
                              att_dst[0].reshape(H, 1))
  h_in = h0
  for l in range(2):
    mvec = jnp.full((16,), _shift_const(mx), f32)
    o2, sb2 = _sc_layer(idx3, asv.reshape(NP), adv.reshape(NP), mvec, hw)
    if l == 0:
      h_in, hw, asv, adv, mx = _tc2(o2[0], o2[1], sb2[0], sb2[1],
                                    bg[0].reshape(1, H), h_in, Wg[1],
                                    att_src[1].reshape(H, 1),
                                    att_dst[1].reshape(H, 1))
    else:
      _, out = _tc3(o2[0], o2[1], sb2[0], sb2[1], bg[1].reshape(1, H), h_in,
                    batchp, W2, b2.reshape(1, OUT))
  return out


# speedup vs baseline: 38.9254x; 1.0859x over previous
import jax
import jax.numpy as jnp
from jax import lax
from jax.experimental import pallas as pl
from jax.experimental.pallas import tpu as pltpu
from jax.experimental.pallas import tpu_sc as plsc

N = 10000
D = 128
H = 128
OUT = 128
G = 64
NP = 10240
NPT = NP // 16
EE = 320000 + N
NCHUNK = 82
EROWS = 32 * NCHUNK
EEP = EROWS * 128
PAD = EEP - EE
RB = 10
_HI = jax.lax.Precision.HIGHEST


def _dot(a, b):
  return jax.lax.dot(a, b, precision=_HI, preferred_element_type=jnp.float32)


def _tc1_body(x_ref, w1_ref, b1_ref, wg_ref, asrc_ref, adst_ref,
              h0_ref, hw_ref, asv_ref, adv_ref, mx_ref):
  h0 = _dot(x_ref[...], w1_ref[...]) + b1_ref[...]
  hw = _dot(h0, wg_ref[...])
  asv = _dot(hw, asrc_ref[...])
  adv = _dot(hw, adst_ref[...])
  h0_ref[...] = h0
  hw_ref[...] = hw
  asv_ref[...] = asv
  adv_ref[...] = adv
  ma = jnp.broadcast_to(jnp.max(asv), (1, 1, 64))
  mb = jnp.broadcast_to(jnp.max(adv), (1, 1, 64))
  mx_ref[...] = jnp.concatenate([ma, mb], axis=2)


_tc1 = pl.pallas_call(
    _tc1_body,
    grid=(RB,),
    in_specs=[
        pl.BlockSpec((1024, D), lambda i: (i, 0)),
        pl.BlockSpec((D, H), lambda i: (0, 0)),
        pl.BlockSpec((1, H), lambda i: (0, 0)),
        pl.BlockSpec((H, H), lambda i: (0, 0)),
        pl.BlockSpec((H, 1), lambda i: (0, 0)),
        pl.BlockSpec((H, 1), lambda i: (0, 0)),
    ],
    out_specs=[
        pl.BlockSpec((1024, H), lambda i: (i, 0)),
        pl.BlockSpec((1024, H), lambda i: (i, 0)),
        pl.BlockSpec((1024, 1), lambda i: (i, 0)),
        pl.BlockSpec((1024, 1), lambda i: (i, 0)),
        pl.BlockSpec((1, 1, 128), lambda i: (i, 0, 0)),
    ],
    out_shape=[
        jax.ShapeDtypeStruct((NP, H), jnp.float32),
        jax.ShapeDtypeStruct((NP, H), jnp.float32),
        jax.ShapeDtypeStruct((NP, 1), jnp.float32),
        jax.ShapeDtypeStruct((NP, 1), jnp.float32),
        jax.ShapeDtypeStruct((RB, 1, 128), jnp.float32),
    ],
)


def _tc2_body(o0_ref, o1_ref, sb0_ref, sb1_ref, bgl_ref, hin_ref, wg_ref,
              asrc_ref, adst_ref, h1_ref, hw_ref, asv_ref, adv_ref, mx_ref):
  denom = sb0_ref[...] + sb1_ref[...] + 1e-16
  agg = (o0_ref[...] + o1_ref[...]) / denom + bgl_ref[...]
  h1 = jnp.maximum(agg, 0.0) + hin_ref[...]
  hw = _dot(h1, wg_ref[...])
  asv = _dot(hw, asrc_ref[...])
  adv = _dot(hw, adst_ref[...])
  h1_ref[...] = h1
  hw_ref[...] = hw
  asv_ref[...] = asv
  adv_ref[...] = adv
  ma = jnp.broadcast_to(jnp.max(asv), (1, 1, 64))
  mb = jnp.broadcast_to(jnp.max(adv), (1, 1, 64))
  mx_ref[...] = jnp.concatenate([ma, mb], axis=2)


_tc2 = pl.pallas_call(
    _tc2_body,
    grid=(RB,),
    in_specs=[
        pl.BlockSpec((1024, H), lambda i: (i, 0)),
        pl.BlockSpec((1024, H), lambda i: (i, 0)),
        pl.BlockSpec((1024, 1), lambda i: (i, 0)),
        pl.BlockSpec((1024, 1), lambda i: (i, 0)),
        pl.BlockSpec((1, H), lambda i: (0, 0)),
        pl.BlockSpec((1024, H), lambda i: (i, 0)),
        pl.BlockSpec((H, H), lambda i: (0, 0)),
        pl.BlockSpec((H, 1), lambda i: (0, 0)),
        pl.BlockSpec((H, 1), lambda i: (0, 0)),
    ],
    out_specs=[
        pl.BlockSpec((1024, H), lambda i: (i, 0)),
        pl.BlockSpec((1024, H), lambda i: (i, 0)),
        pl.BlockSpec((1024, 1), lambda i: (i, 0)),
        pl.BlockSpec((1024, 1), lambda i: (i, 0)),
        pl.BlockSpec((1, 1, 128), lambda i: (i, 0, 0)),
    ],
    out_shape=[
        jax.ShapeDtypeStruct((NP, H), jnp.float32),
        jax.ShapeDtypeStruct((NP, H), jnp.float32),
        jax.ShapeDtypeStruct((NP, 1), jnp.float32),
        jax.ShapeDtypeStruct((NP, 1), jnp.float32),
        jax.ShapeDtypeStruct((RB, 1, 128), jnp.float32),
    ],
)


def _tc3_body(o0_ref, o1_ref, sb0_ref, sb1_ref, bgl_ref, hin_ref, batch_ref,
              w2_ref, b2_ref, pooled_ref, out_ref):
  i = pl.program_id(0)
  denom = sb0_ref[...] + sb1_ref[...] + 1e-16
  agg = (o0_ref[...] + o1_ref[...]) / denom + bgl_ref[...]
  h2 = jnp.maximum(agg, 0.0) + hin_ref[...]
  bt = batch_ref[...].reshape(1, 1024)
  gi = jax.lax.broadcasted_iota(jnp.int32, (G, 1024), 0)
  onehot = (gi == bt).astype(jnp.float32)
  part = _dot(onehot, h2)

  @pl.when(i == 0)
  def _():
    pooled_ref[...] = part

  @pl.when(i > 0)
  def _():
    pooled_ref[...] = pooled_ref[...] + part

  @pl.when(i == RB - 1)
  def _():
    out_ref[...] = _dot(pooled_ref[...], w2_ref[...]) + b2_ref[...]


_tc3 = pl.pallas_call(
    _tc3_body,
    grid=(RB,),
    in_specs=[
        pl.BlockSpec((1024, H), lambda i: (i, 0)),
        pl.BlockSpec((1024, H), lambda i: (i, 0)),
        pl.BlockSpec((1024, 1), lambda i: (i, 0)),
        pl.BlockSpec((1024, 1), lambda i: (i, 0)),
        pl.BlockSpec((1, H), lambda i: (0, 0)),
        pl.BlockSpec((1024, H), lambda i: (i, 0)),
        pl.BlockSpec((1024,), lambda i: (i,)),
        pl.BlockSpec((H, OUT), lambda i: (0, 0)),
        pl.BlockSpec((1, OUT), lambda i: (0, 0)),
    ],
    out_specs=[
        pl.BlockSpec((G, H), lambda i: (0, 0)),
        pl.BlockSpec((G, OUT), lambda i: (0, 0)),
    ],
    out_shape=[
        jax.ShapeDtypeStruct((G, H), jnp.float32),
        jax.ShapeDtypeStruct((G, OUT), jnp.float32),
    ],
)


def _sc_layer_body(idx3_h, asv_h, adv_h, mvec_h, hw_h,
                   o_out, sb_out,
                   idx_a, idx_b, asg_a, asg_b, adg_a, adg_b, w_a, w_b,
                   rows_a, rows_b, sv_t, m_t, s_sh, o_sh,
                   sg_a, sg_b, sr_a, sr_b, so_a, so_b):
  cid = lax.axis_index("c")
  sid = lax.axis_index("s")
  z16 = jnp.zeros((16,), jnp.float32)
  tb = (cid * 16 + sid) * NCHUNK

  def prefetch(c, idx, asg, adg, rows, sg, sr):
    pltpu.sync_copy(idx3_h.at[c], idx)
    pltpu.async_copy(asv_h.at[idx.at[0]], asg, sg)
    pltpu.async_copy(adv_h.at[idx.at[1]], adg, sg)
    pltpu.async_copy(hw_h.at[idx.at[0]], rows, sr)

  def process(idx, asg, adg, w_t, rows, sg, sr, so):
    pltpu.make_async_copy(asv_h.at[idx.at[0]], asg, sg).wait()
    pltpu.make_async_copy(adv_h.at[idx.at[1]], adg, sg).wait()
    m16 = m_t[...]
    for i in range(8):
      sl = pl.ds(16 * i, 16)
      t = asg[sl] + adg[sl]
      e = jnp.where(t > 0, t, 0.2 * t)
      w_t[sl] = jnp.exp(e - m16)
    pltpu.make_async_copy(hw_h.at[idx.at[0]], rows, sr).wait()

    @pl.loop(0, 128, unroll=8)
    def _scale(j):
      wb = plsc.load_gather(w_t, [jnp.full((16,), j, jnp.int32)])
      for k in range(8):
        sl = pl.ds(16 * k, 16)
        rows[j, sl] = rows[j, sl] * wb

    pltpu.async_copy(w_t, s_sh.at[idx.at[1]], so, add=True)
    pltpu.async_copy(rows, o_sh.at[idx.at[1]], so, add=True)

  def drain(idx, w_t, rows, so):
    pltpu.make_async_copy(w_t, s_sh.at[idx.at[1]], so).wait()
    pltpu.make_async_copy(rows, o_sh.at[idx.at[1]], so).wait()

  @pl.loop(0, 128, unroll=4)
  def _zrows(j):
    for k in range(8):
      rows_b[j, pl.ds(16 * k, 16)] = z16

  for j in range(NPT // 16):
    sv_t[pl.ds(16 * j, 16)] = z16
  pltpu.sync_copy(mvec_h, m_t)
  prefetch(tb, idx_a, asg_a, adg_a, rows_a, sg_a, sr_a)
  for b in range(NPT // 128):
    pltpu.sync_copy(rows_b, o_sh.at[pl.ds(sid * NPT + b * 128, 128)])
  pltpu.sync_copy(sv_t, s_sh.at[pl.ds(sid * NPT, NPT)])
  plsc.subcore_barrier()

  @pl.loop(0, NCHUNK // 2)
  def _pair(t):
    c0 = tb + 2 * t

    @pl.when(t > 0)
    def _():
      drain(idx_b, w_b, rows_b, so_b)

    prefetch(c0 + 1, idx_b, asg_b, adg_b, rows_b, sg_b, sr_b)
    process(idx_a, asg_a, adg_a, w_a, rows_a, sg_a, sr_a, so_a)
    process(idx_b, asg_b, adg_b, w_b, rows_b, sg_b, sr_b, so_b)

    @pl.when(t < NCHUNK // 2 - 1)
    def _():
      drain(idx_a, w_a, rows_a, so_a)
      prefetch(c0 + 2, idx_a, asg_a, adg_a, rows_a, sg_a, sr_a)

  drain(idx_a, w_a, rows_a, so_a)
  drain(idx_b, w_b, rows_b, so_b)
  plsc.subcore_barrier()

  pltpu.sync_copy(o_sh.at[pl.ds(sid * NPT, NPT)],
                  o_out.at[cid, pl.ds(sid * NPT, NPT)])
  pltpu.sync_copy(s_sh.at[pl.ds(sid * NPT, NPT)],
                  sb_out.at[pl.ds(cid * NP + sid * NPT, NPT)])


_sc_layer = pl.kernel(
    _sc_layer_body,
    out_type=(
        jax.ShapeDtypeStruct((2, NP, H), jnp.float32),
        jax.ShapeDtypeStruct((2 * NP,), jnp.float32),
    ),
    mesh=plsc.VectorSubcoreMesh(core_axis_name="c", subcore_axis_name="s",
                                num_cores=2, num_subcores=16),
    compiler_params=pltpu.CompilerParams(needs_layout_passes=False),
    scratch_types=[
        pltpu.VMEM((2, 128), jnp.int32),
        pltpu.VMEM((2, 128), jnp.int32),
        pltpu.VMEM((128,), jnp.float32),
        pltpu.VMEM((128,), jnp.float32),
        pltpu.VMEM((128,), jnp.float32),
        pltpu.VMEM((128,), jnp.float32),
        pltpu.VMEM((128,), jnp.float32),
        pltpu.VMEM((128,), jnp.float32),
        pltpu.VMEM((128, H), jnp.float32),
        pltpu.VMEM((128, H), jnp.float32),
        pltpu.VMEM((NPT,), jnp.float32),
        pltpu.VMEM((16,), jnp.float32),
        pltpu.VMEM_SHARED((NP,), jnp.float32),
        pltpu.VMEM_SHARED((NP, H), jnp.float32),
        pltpu.SemaphoreType.DMA,
        pltpu.SemaphoreType.DMA,
        pltpu.SemaphoreType.DMA,
        pltpu.SemaphoreType.DMA,
        pltpu.SemaphoreType.DMA,
        pltpu.SemaphoreType.DMA,
    ],
)


def _shift_const(mx):
  tmax = jnp.max(mx[:, 0, :64]) + jnp.max(mx[:, 0, 64:])
  return jnp.where(tmax > 0, tmax, 0.2 * tmax)


def kernel(x, edge_index, batch, W1, b1, Wg, att_src, att_dst, bg, W2, b2):
  f32 = jnp.float32
  xp = jnp.zeros((NP, D), f32).at[:N].set(x)
  loops = jnp.arange(N, dtype=jnp.int32)
  pad_src = (jnp.arange(PAD, dtype=jnp.int32) * 37) % N
  pad_dst = N + (jnp.arange(PAD, dtype=jnp.int32) % (NP - N))
  src_f = jnp.concatenate(
      [edge_index[0].astype(jnp.int32), loops, pad_src]).reshape(EROWS, 1, 128)
  dst_f = jnp.concatenate(
      [edge_index[1].astype(jnp.int32), loops, pad_dst]).reshape(EROWS, 1, 128)
  idx3 = jnp.concatenate([src_f, dst_f], axis=1)
  batchp = jnp.full((NP,), G, jnp.int32).at[:N].set(batch.astype(jnp.int32))

  h0, hw, asv, adv, mx = _tc1(xp, W1, b1.reshape(1, H), Wg[0],
                              att_src[0].reshape(H, 1),
                              att_dst[0].reshape(H, 1))
  h_in = h0
  for l in range(2):
    mvec = jnp.full((16,), _shift_const(mx), f32)
    o2, s2 = _sc_layer(idx3, asv.reshape(NP), adv.reshape(NP), mvec, hw)
    sb2 = s2.reshape(2, NP, 1)
    if l == 0:
      h_in, hw, asv, adv, mx = _tc2(o2[0], o2[1], sb2[0], sb2[1],
                                    bg[0].reshape(1, H), h_in, Wg[1],
                                    att_src[1].reshape(H, 1),
                                    att_dst[1].reshape(H, 1))
    else:
      _, out = _tc3(o2[0], o2[1], sb2[0], sb2[1], bg[1].reshape(1, H), h_in,
                    batchp, W2, b2.reshape(1, OUT))
  return out


# speedup vs baseline: 42.3071x; 1.0869x over previous
import jax
import jax.numpy as jnp
from jax import lax
from jax.experimental import pallas as pl
from jax.experimental.pallas import tpu as pltpu
from jax.experimental.pallas import tpu_sc as plsc

N = 10000
D = 128
H = 128
OUT = 128
G = 64
NP = 10240
NPT = NP // 16
EE = 320000 + N
NCHUNK = 82
EROWS = 32 * NCHUNK
EEP = EROWS * 128
PAD = EEP - EE
RB = 10
_HI = jax.lax.Precision.HIGHEST


def _dot(a, b):
  return jax.lax.dot(a, b, precision=_HI, preferred_element_type=jnp.float32)


def _tc1_body(x_ref, w1_ref, b1_ref, wg_ref, asrc_ref, adst_ref,
              h0_ref, hw_ref, asv_ref, adv_ref, mx_ref):
  h0 = _dot(x_ref[...], w1_ref[...]) + b1_ref[...]
  hw = _dot(h0, wg_ref[...])
  asv = _dot(hw, asrc_ref[...])
  adv = _dot(hw, adst_ref[...])
  h0_ref[...] = h0
  hw_ref[...] = hw
  asv_ref[...] = asv
  adv_ref[...] = adv
  ma = jnp.broadcast_to(jnp.max(asv), (1, 1, 64))
  mb = jnp.broadcast_to(jnp.max(adv), (1, 1, 64))
  mx_ref[...] = jnp.concatenate([ma, mb], axis=2)


_tc1 = pl.pallas_call(
    _tc1_body,
    grid=(RB,),
    in_specs=[
        pl.BlockSpec((1024, D), lambda i: (i, 0)),
        pl.BlockSpec((D, H), lambda i: (0, 0)),
        pl.BlockSpec((1, H), lambda i: (0, 0)),
        pl.BlockSpec((H, H), lambda i: (0, 0)),
        pl.BlockSpec((H, 1), lambda i: (0, 0)),
        pl.BlockSpec((H, 1), lambda i: (0, 0)),
    ],
    out_specs=[
        pl.BlockSpec((1024, H), lambda i: (i, 0)),
        pl.BlockSpec((1024, H), lambda i: (i, 0)),
        pl.BlockSpec((1024, 1), lambda i: (i, 0)),
        pl.BlockSpec((1024, 1), lambda i: (i, 0)),
        pl.BlockSpec((1, 1, 128), lambda i: (i, 0, 0)),
    ],
    out_shape=[
        jax.ShapeDtypeStruct((NP, H), jnp.float32),
        jax.ShapeDtypeStruct((NP, H), jnp.float32),
        jax.ShapeDtypeStruct((NP, 1), jnp.float32),
        jax.ShapeDtypeStruct((NP, 1), jnp.float32),
        jax.ShapeDtypeStruct((RB, 1, 128), jnp.float32),
    ],
)


def _tc2_body(o0_ref, o1_ref, sb0_ref, sb1_ref, bgl_ref, hin_ref, wg_ref,
              asrc_ref, adst_ref, h1_ref, hw_ref, asv_ref, adv_ref, mx_ref):
  denom = sb0_ref[...] + sb1_ref[...] + 1e-16
  agg = (o0_ref[...] + o1_ref[...]) / denom + bgl_ref[...]
  h1 = jnp.maximum(agg, 0.0) + hin_ref[...]
  hw = _dot(h1, wg_ref[...])
  asv = _dot(hw, asrc_ref[...])
  adv = _dot(hw, adst_ref[...])
  h1_ref[...] = h1
  hw_ref[...] = hw
  asv_ref[...] = asv
  adv_ref[...] = adv
  ma = jnp.broadcast_to(jnp.max(asv), (1, 1, 64))
  mb = jnp.broadcast_to(jnp.max(adv), (1, 1, 64))
  mx_ref[...] = jnp.concatenate([ma, mb], axis=2)


_tc2 = pl.pallas_call(
    _tc2_body,
    grid=(RB,),
    in_specs=[
        pl.BlockSpec((1024, H), lambda i: (i, 0)),
        pl.BlockSpec((1024, H), lambda i: (i, 0)),
        pl.BlockSpec((1024, 1), lambda i: (i, 0)),
        pl.BlockSpec((1024, 1), lambda i: (i, 0)),
        pl.BlockSpec((1, H), lambda i: (0, 0)),
        pl.BlockSpec((1024, H), lambda i: (i, 0)),
        pl.BlockSpec((H, H), lambda i: (0, 0)),
        pl.BlockSpec((H, 1), lambda i: (0, 0)),
        pl.BlockSpec((H, 1), lambda i: (0, 0)),
    ],
    out_specs=[
        pl.BlockSpec((1024, H), lambda i: (i, 0)),
        pl.BlockSpec((1024, H), lambda i: (i, 0)),
        pl.BlockSpec((1024, 1), lambda i: (i, 0)),
        pl.BlockSpec((1024, 1), lambda i: (i, 0)),
        pl.BlockSpec((1, 1, 128), lambda i: (i, 0, 0)),
    ],
    out_shape=[
        jax.ShapeDtypeStruct((NP, H), jnp.float32),
        jax.ShapeDtypeStruct((NP, H), jnp.float32),
        jax.ShapeDtypeStruct((NP, 1), jnp.float32),
        jax.ShapeDtypeStruct((NP, 1), jnp.float32),
        jax.ShapeDtypeStruct((RB, 1, 128), jnp.float32),
    ],
)


def _tc3_body(o0_ref, o1_ref, sb0_ref, sb1_ref, bgl_ref, hin_ref, batch_ref,
              w2_ref, b2_ref, pooled_ref, out_ref):
  i = pl.program_id(0)
  denom = sb0_ref[...] + sb1_ref[...] + 1e-16
  agg = (o0_ref[...] + o1_ref[...]) / denom + bgl_ref[...]
  h2 = jnp.maximum(agg, 0.0) + hin_ref[...]
  bt = batch_ref[...].reshape(1, 1024)
  gi = jax.lax.broadcasted_iota(jnp.int32, (G, 1024), 0)
  onehot = (gi == bt).astype(jnp.float32)
  part = _dot(onehot, h2)

  @pl.when(i == 0)
  def _():
    pooled_ref[...] = part

  @pl.when(i > 0)
  def _():
    pooled_ref[...] = pooled_ref[...] + part

  @pl.when(i == RB - 1)
  def _():
    out_ref[...] = _dot(pooled_ref[...], w2_ref[...]) + b2_ref[...]


_tc3 = pl.pallas_call(
    _tc3_body,
    grid=(RB,),
    in_specs=[
        pl.BlockSpec((1024, H), lambda i: (i, 0)),
        pl.BlockSpec((1024, H), lambda i: (i, 0)),
        pl.BlockSpec((1024, 1), lambda i: (i, 0)),
        pl.BlockSpec((1024, 1), lambda i: (i, 0)),
        pl.BlockSpec((1, H), lambda i: (0, 0)),
        pl.BlockSpec((1024, H), lambda i: (i, 0)),
        pl.BlockSpec((1024,), lambda i: (i,)),
        pl.BlockSpec((H, OUT), lambda i: (0, 0)),
        pl.BlockSpec((1, OUT), lambda i: (0, 0)),
    ],
    out_specs=[
        pl.BlockSpec((G, H), lambda i: (0, 0)),
        pl.BlockSpec((G, OUT), lambda i: (0, 0)),
    ],
    out_shape=[
        jax.ShapeDtypeStruct((G, H), jnp.float32),
        jax.ShapeDtypeStruct((G, OUT), jnp.float32),
    ],
)


def _sc_layer_body(idx3_h, asv_h, adv_h, mvec_h, hw_h,
                   o_out, sb_out,
                   idx_a, idx_b, asg_a, asg_b, adg_a, adg_b, w_a, w_b,
                   rows_a, rows_b, sv_t, m_t, s_sh, o_sh,
                   sg_a, sg_b, sr_a, sr_b, so_a, so_b):
  cid = lax.axis_index("c")
  sid = lax.axis_index("s")
  z16 = jnp.zeros((16,), jnp.float32)
  tb = (cid * 16 + sid) * NCHUNK

  def prefetch(c, idx, asg, adg, rows, sg, sr):
    pltpu.sync_copy(idx3_h.at[c], idx)
    pltpu.async_copy(asv_h.at[idx.at[0]], asg, sg)
    pltpu.async_copy(adv_h.at[idx.at[1]], adg, sg)
    pltpu.async_copy(hw_h.at[idx.at[0]], rows, sr)

  def process(idx, asg, adg, w_t, rows, sg, sr, so):
    pltpu.make_async_copy(asv_h.at[idx.at[0]], asg, sg).wait()
    pltpu.make_async_copy(adv_h.at[idx.at[1]], adg, sg).wait()
    m16 = m_t[...]
    for i in range(8):
      sl = pl.ds(16 * i, 16)
      t = asg[sl] + adg[sl]
      e = jnp.where(t > 0, t, 0.2 * t)
      w_t[sl] = jnp.exp(e - m16)
    pltpu.make_async_copy(hw_h.at[idx.at[0]], rows, sr).wait()

    @pl.loop(0, 8)
    def _scale(g):
      v16 = w_t[pl.ds(g * 16, 16)]
      for j in range(16):
        wb = jnp.full((16,), v16[j], jnp.float32)
        base = g * 16 + j
        for k in range(8):
          sl = pl.ds(16 * k, 16)
          rows[base, sl] = rows[base, sl] * wb

    pltpu.async_copy(w_t, s_sh.at[idx.at[1]], so, add=True)
    pltpu.async_copy(rows, o_sh.at[idx.at[1]], so, add=True)

  def drain(idx, w_t, rows, so):
    pltpu.make_async_copy(w_t, s_sh.at[idx.at[1]], so).wait()
    pltpu.make_async_copy(rows, o_sh.at[idx.at[1]], so).wait()

  @pl.loop(0, 128, unroll=4)
  def _zrows(j):
    for k in range(8):
      rows_b[j, pl.ds(16 * k, 16)] = z16

  for j in range(NPT // 16):
    sv_t[pl.ds(16 * j, 16)] = z16
  pltpu.sync_copy(mvec_h, m_t)
  prefetch(tb, idx_a, asg_a, adg_a, rows_a, sg_a, sr_a)
  for b in range(NPT // 128):
    pltpu.sync_copy(rows_b, o_sh.at[pl.ds(sid * NPT + b * 128, 128)])
  pltpu.sync_copy(sv_t, s_sh.at[pl.ds(sid * NPT, NPT)])
  plsc.subcore_barrier()

  @pl.loop(0, NCHUNK // 2)
  def _pair(t):
    c0 = tb + 2 * t

    @pl.when(t > 0)
    def _():
      drain(idx_b, w_b, rows_b, so_b)

    prefetch(c0 + 1, idx_b, asg_b, adg_b, rows_b, sg_b, sr_b)
    process(idx_a, asg_a, adg_a, w_a, rows_a, sg_a, sr_a, so_a)
    process(idx_b, asg_b, adg_b, w_b, rows_b, sg_b, sr_b, so_b)

    @pl.when(t < NCHUNK // 2 - 1)
    def _():
      drain(idx_a, w_a, rows_a, so_a)
      prefetch(c0 + 2, idx_a, asg_a, adg_a, rows_a, sg_a, sr_a)

  drain(idx_a, w_a, rows_a, so_a)
  drain(idx_b, w_b, rows_b, so_b)
  plsc.subcore_barrier()

  pltpu.sync_copy(o_sh.at[pl.ds(sid * NPT, NPT)],
                  o_out.at[cid, pl.ds(sid * NPT, NPT)])
  pltpu.sync_copy(s_sh.at[pl.ds(sid * NPT, NPT)],
                  sb_out.at[pl.ds(cid * NP + sid * NPT, NPT)])


_sc_layer = pl.kernel(
    _sc_layer_body,
    out_type=(
        jax.ShapeDtypeStruct((2, NP, H), jnp.float32),
        jax.ShapeDtypeStruct((2 * NP,), jnp.float32),
    ),
    mesh=plsc.VectorSubcoreMesh(core_axis_name="c", subcore_axis_name="s",
                                num_cores=2, num_subcores=16),
    compiler_params=pltpu.CompilerParams(needs_layout_passes=False),
    scratch_types=[
        pltpu.VMEM((2, 128), jnp.int32),
        pltpu.VMEM((2, 128), jnp.int32),
        pltpu.VMEM((128,), jnp.float32),
        pltpu.VMEM((128,), jnp.float32),
        pltpu.VMEM((128,), jnp.float32),
        pltpu.VMEM((128,), jnp.float32),
        pltpu.VMEM((128,), jnp.float32),
        pltpu.VMEM((128,), jnp.float32),
        pltpu.VMEM((128, H), jnp.float32),
        pltpu.VMEM((128, H), jnp.float32),
        pltpu.VMEM((NPT,), jnp.float32),
        pltpu.VMEM((16,), jnp.float32),
        pltpu.VMEM_SHARED((NP,), jnp.float32),
        pltpu.VMEM_SHARED((NP, H), jnp.float32),
        pltpu.SemaphoreType.DMA,
        pltpu.SemaphoreType.DMA,
        pltpu.SemaphoreType.DMA,
        pltpu.SemaphoreType.DMA,
        pltpu.SemaphoreType.DMA,
        pltpu.SemaphoreType.DMA,
    ],
)


def _shift_const(mx):
  tmax = jnp.max(mx[:, 0, :64]) + jnp.max(mx[:, 0, 64:])
  return jnp.where(tmax > 0, tmax, 0.2 * tmax)


def kernel(x, edge_index, batch, W1, b1, Wg, att_src, att_dst, bg, W2, b2):
  f32 = jnp.float32
  xp = jnp.zeros((NP, D), f32).at[:N].set(x)
  loops = jnp.arange(N, dtype=jnp.int32)
  pad_src = (jnp.arange(PAD, dtype=jnp.int32) * 37) % N
  pad_dst = N + (jnp.arange(PAD, dtype=jnp.int32) % (NP - N))
  src_f = jnp.concatenate(
      [edge_index[0].astype(jnp.int32), loops, pad_src]).reshape(EROWS, 1, 128)
  dst_f = jnp.concatenate(
      [edge_index[1].astype(jnp.int32), loops, pad_dst]).reshape(EROWS, 1, 128)
  idx3 = jnp.concatenate([src_f, dst_f], axis=1)
  batchp = jnp.full((NP,), G, jnp.int32).at[:N].set(batch.astype(jnp.int32))

  h0, hw, asv, adv, mx = _tc1(xp, W1, b1.reshape(1, H), Wg[0],
                              att_src[0].reshape(H, 1),
                              att_dst[0].reshape(H, 1))
  h_in = h0
  for l in range(2):
    mvec = jnp.full((16,), _shift_const(mx), f32)
    o2, s2 = _sc_layer(idx3, asv.reshape(NP), adv.reshape(NP), mvec, hw)
    sb2 = s2.reshape(2, NP, 1)
    if l == 0:
      h_in, hw, asv, adv, mx = _tc2(o2[0], o2[1], sb2[0], sb2[1],
                                    bg[0].reshape(1, H), h_in, Wg[1],
                                    att_src[1].reshape(H, 1),
                                    att_dst[1].reshape(H, 1))
    else:
      _, out = _tc3(o2[0], o2[1], sb2[0], sb2[1], bg[1].reshape(1, H), h_in,
                    batchp, W2, b2.reshape(1, OUT))
  return out


# speedup vs baseline: 43.9200x; 1.0381x over previous
import jax
import jax.numpy as jnp
from jax import lax
from jax.experimental import pallas as pl
from jax.experimental.pallas import tpu as pltpu
from jax.experimental.pallas import tpu_sc as plsc

N = 10000
D = 128
H = 128
OUT = 128
G = 64
NP = 10240
NPT = NP // 16
EE = 320000 + N
NCHUNK = 82
EROWS = 32 * NCHUNK
EEP = EROWS * 128
PAD = EEP - EE
RB = 10
_HI = jax.lax.Precision.HIGHEST


def _dot(a, b):
  return jax.lax.dot(a, b, precision=_HI, preferred_element_type=jnp.float32)


def _tc1_body(x_ref, w1_ref, b1_ref, wg_ref, asrc_ref, adst_ref,
              h0_ref, hw_ref, asv_ref, adv_ref, mx_ref):
  h0 = _dot(x_ref[...], w1_ref[...]) + b1_ref[...]
  hw = _dot(h0, wg_ref[...])
  asv = _dot(hw, asrc_ref[...])
  adv = _dot(hw, adst_ref[...])
  h0_ref[...] = h0
  hw_ref[...] = hw
  asv_ref[...] = asv
  adv_ref[...] = adv
  ma = jnp.broadcast_to(jnp.max(asv), (1, 1, 64))
  mb = jnp.broadcast_to(jnp.max(adv), (1, 1, 64))
  mx_ref[...] = jnp.concatenate([ma, mb], axis=2)


_tc1 = pl.pallas_call(
    _tc1_body,
    grid=(RB,),
    in_specs=[
        pl.BlockSpec((1024, D), lambda i: (i, 0)),
        pl.BlockSpec((D, H), lambda i: (0, 0)),
        pl.BlockSpec((1, H), lambda i: (0, 0)),
        pl.BlockSpec((H, H), lambda i: (0, 0)),
        pl.BlockSpec((H, 1), lambda i: (0, 0)),
        pl.BlockSpec((H, 1), lambda i: (0, 0)),
    ],
    out_specs=[
        pl.BlockSpec((1024, H), lambda i: (i, 0)),
        pl.BlockSpec((1024, H), lambda i: (i, 0)),
        pl.BlockSpec((1024, 1), lambda i: (i, 0)),
        pl.BlockSpec((1024, 1), lambda i: (i, 0)),
        pl.BlockSpec((1, 1, 128), lambda i: (i, 0, 0)),
    ],
    out_shape=[
        jax.ShapeDtypeStruct((NP, H), jnp.float32),
        jax.ShapeDtypeStruct((NP, H), jnp.float32),
        jax.ShapeDtypeStruct((NP, 1), jnp.float32),
        jax.ShapeDtypeStruct((NP, 1), jnp.float32),
        jax.ShapeDtypeStruct((RB, 1, 128), jnp.float32),
    ],
)


def _tc2_body(o0_ref, o1_ref, sb0_ref, sb1_ref, bgl_ref, hin_ref, wg_ref,
              asrc_ref, adst_ref, h1_ref, hw_ref, asv_ref, adv_ref, mx_ref):
  denom = sb0_ref[...] + sb1_ref[...] + 1e-16
  agg = (o0_ref[...] + o1_ref[...]) / denom + bgl_ref[...]
  h1 = jnp.maximum(agg, 0.0) + hin_ref[...]
  hw = _dot(h1, wg_ref[...])
  asv = _dot(hw, asrc_ref[...])
  adv = _dot(hw, adst_ref[...])
  h1_ref[...] = h1
  hw_ref[...] = hw
  asv_ref[...] = asv
  adv_ref[...] = adv
  ma = jnp.broadcast_to(jnp.max(asv), (1, 1, 64))
  mb = jnp.broadcast_to(jnp.max(adv), (1, 1, 64))
  mx_ref[...] = jnp.concatenate([ma, mb], axis=2)


_tc2 = pl.pallas_call(
    _tc2_body,
    grid=(RB,),
    in_specs=[
        pl.BlockSpec((1024, H), lambda i: (i, 0)),
        pl.BlockSpec((1024, H), lambda i: (i, 0)),
        pl.BlockSpec((1024, 1), lambda i: (i, 0)),
        pl.BlockSpec((1024, 1), lambda i: (i, 0)),
        pl.BlockSpec((1, H), lambda i: (0, 0)),
        pl.BlockSpec((1024, H), lambda i: (i, 0)),
        pl.BlockSpec((H, H), lambda i: (0, 0)),
        pl.BlockSpec((H, 1), lambda i: (0, 0)),
        pl.BlockSpec((H, 1), lambda i: (0, 0)),
    ],
    out_specs=[
        pl.BlockSpec((1024, H), lambda i: (i, 0)),
        pl.BlockSpec((1024, H), lambda i: (i, 0)),
        pl.BlockSpec((1024, 1), lambda i: (i, 0)),
        pl.BlockSpec((1024, 1), lambda i: (i, 0)),
        pl.BlockSpec((1, 1, 128), lambda i: (i, 0, 0)),
    ],
    out_shape=[
        jax.ShapeDtypeStruct((NP, H), jnp.float32),
        jax.ShapeDtypeStruct((NP, H), jnp.float32),
        jax.ShapeDtypeStruct((NP, 1), jnp.float32),
        jax.ShapeDtypeStruct((NP, 1), jnp.float32),
        jax.ShapeDtypeStruct((RB, 1, 128), jnp.float32),
    ],
)


def _tc3_body(o0_ref, o1_ref, sb0_ref, sb1_ref, bgl_ref, hin_ref, batch_ref,
              w2_ref, b2_ref, pooled_ref, out_ref):
  i = pl.program_id(0)
  denom = sb0_ref[...] + sb1_ref[...] + 1e-16
  agg = (o0_ref[...] + o1_ref[...]) / denom + bgl_ref[...]
  h2 = jnp.maximum(agg, 0.0) + hin_ref[...]
  bt = batch_ref[...].reshape(1, 1024)
  gi = jax.lax.broadcasted_iota(jnp.int32, (G, 1024), 0)
  onehot = (gi == bt).astype(jnp.float32)
  part = _dot(onehot, h2)

  @pl.when(i == 0)
  def _():
    pooled_ref[...] = part

  @pl.when(i > 0)
  def _():
    pooled_ref[...] = pooled_ref[...] + part

  @pl.when(i == RB - 1)
  def _():
    out_ref[...] = _dot(pooled_ref[...], w2_ref[...]) + b2_ref[...]


_tc3 = pl.pallas_call(
    _tc3_body,
    grid=(RB,),
    in_specs=[
        pl.BlockSpec((1024, H), lambda i: (i, 0)),
        pl.BlockSpec((1024, H), lambda i: (i, 0)),
        pl.BlockSpec((1024, 1), lambda i: (i, 0)),
        pl.BlockSpec((1024, 1), lambda i: (i, 0)),
        pl.BlockSpec((1, H), lambda i: (0, 0)),
        pl.BlockSpec((1024, H), lambda i: (i, 0)),
        pl.BlockSpec((1024,), lambda i: (i,)),
        pl.BlockSpec((H, OUT), lambda i: (0, 0)),
        pl.BlockSpec((1, OUT), lambda i: (0, 0)),
    ],
    out_specs=[
        pl.BlockSpec((G, H), lambda i: (0, 0)),
        pl.BlockSpec((G, OUT), lambda i: (0, 0)),
    ],
    out_shape=[
        jax.ShapeDtypeStruct((G, H), jnp.float32),
        jax.ShapeDtypeStruct((G, OUT), jnp.float32),
    ],
)


def _sc_layer_body(idx3_h, asv_h, adv_h, mvec_h, hw_h,
                   o_out, sb_out,
                   idx_a, idxp, asg_a, asg_b, adg_a, adg_b, w_a, w_b,
                   rows_a, rows_b, sv_t, m_t, s_sh, o_sh,
                   sg_a, sg_b, sr_a, sr_b, so_a, so_b):
  cid = lax.axis_index("c")
  sid = lax.axis_index("s")
  z16 = jnp.zeros((16,), jnp.float32)
  tb = (cid * 16 + sid) * NCHUNK

  def prefetch(c, idx, asg, adg, rows, sg, sr):
    pltpu.sync_copy(idx3_h.at[c], idx)
    pltpu.async_copy(asv_h.at[idx.at[0]], asg, sg)
    pltpu.async_copy(adv_h.at[idx.at[1]], adg, sg)
    pltpu.async_copy(hw_h.at[idx.at[0]], rows, sr)

  def process(src_i, dst_i, asg, adg, w_t, rows, sg, sr, so):
    pltpu.make_async_copy(asv_h.at[src_i], asg, sg).wait()
    pltpu.make_async_copy(adv_h.at[dst_i], adg, sg).wait()
    m16 = m_t[...]
    for i in range(8):
      sl = pl.ds(16 * i, 16)
      t = asg[sl] + adg[sl]
      e = jnp.where(t > 0, t, 0.2 * t)
      w_t[sl] = jnp.exp(e - m16)
    pltpu.make_async_copy(hw_h.at[src_i], rows, sr).wait()

    @pl.loop(0, 8)
    def _scale(g):
      v16 = w_t[pl.ds(g * 16, 16)]
      for j in range(16):
        wb = jnp.full((16,), v16[j], jnp.float32)
        base = g * 16 + j
        for k in range(8):
          sl = pl.ds(16 * k, 16)
          rows[base, sl] = rows[base, sl] * wb

    pltpu.async_copy(w_t, s_sh.at[dst_i], so, add=True)
    pltpu.async_copy(rows, o_sh.at[dst_i], so, add=True)

  def drain(dst_i, w_t, rows, so):
    pltpu.make_async_copy(w_t, s_sh.at[dst_i], so).wait()
    pltpu.make_async_copy(rows, o_sh.at[dst_i], so).wait()

  @pl.loop(0, 128, unroll=4)
  def _zrows(j):
    for k in range(8):
      rows_b[j, pl.ds(16 * k, 16)] = z16

  for j in range(NPT // 16):
    sv_t[pl.ds(16 * j, 16)] = z16
  pltpu.sync_copy(mvec_h, m_t)
  prefetch(tb, idx_a, asg_a, adg_a, rows_a, sg_a, sr_a)
  for b in range(NPT // 128):
    pltpu.sync_copy(rows_b, o_sh.at[pl.ds(sid * NPT + b * 128, 128)])
  pltpu.sync_copy(sv_t, s_sh.at[pl.ds(sid * NPT, NPT)])
  plsc.subcore_barrier()

  @pl.loop(0, NCHUNK // 2)
  def _pair(t):
    c0 = tb + 2 * t

    @pl.when(t > 0)
    def _():
      drain(idxp.at[0, 1], w_b, rows_b, so_b)

    pltpu.sync_copy(idx3_h.at[pl.ds(c0 + 1, 2)], idxp)
    pltpu.async_copy(asv_h.at[idxp.at[0, 0]], asg_b, sg_b)
    pltpu.async_copy(adv_h.at[idxp.at[0, 1]], adg_b, sg_b)
    pltpu.async_copy(hw_h.at[idxp.at[0, 0]], rows_b, sr_b)
    process(idx_a.at[0], idx_a.at[1], asg_a, adg_a, w_a, rows_a,
            sg_a, sr_a, so_a)
    process(idxp.at[0, 0], idxp.at[0, 1], asg_b, adg_b, w_b, rows_b,
            sg_b, sr_b, so_b)

    @pl.when(t < NCHUNK // 2 - 1)
    def _():
      drain(idx_a.at[1], w_a, rows_a, so_a)
      for r in range(2):
        for i in range(8):
          sl = pl.ds(16 * i, 16)
          idx_a[r, sl] = idxp[1, r, sl]
      pltpu.async_copy(asv_h.at[idx_a.at[0]], asg_a, sg_a)
      pltpu.async_copy(adv_h.at[idx_a.at[1]], adg_a, sg_a)
      pltpu.async_copy(hw_h.at[idx_a.at[0]], rows_a, sr_a)

  drain(idx_a.at[1], w_a, rows_a, so_a)
  drain(idxp.at[0, 1], w_b, rows_b, so_b)
  plsc.subcore_barrier()

  pltpu.sync_copy(o_sh.at[pl.ds(sid * NPT, NPT)],
                  o_out.at[cid, pl.ds(sid * NPT, NPT)])
  pltpu.sync_copy(s_sh.at[pl.ds(sid * NPT, NPT)],
                  sb_out.at[pl.ds(cid * NP + sid * NPT, NPT)])


_sc_layer = pl.kernel(
    _sc_layer_body,
    out_type=(
        jax.ShapeDtypeStruct((2, NP, H), jnp.float32),
        jax.ShapeDtypeStruct((2 * NP,), jnp.float32),
    ),
    mesh=plsc.VectorSubcoreMesh(core_axis_name="c", subcore_axis_name="s",
                                num_cores=2, num_subcores=16),
    compiler_params=pltpu.CompilerParams(needs_layout_passes=False),
    scratch_types=[
        pltpu.VMEM((2, 128), jnp.int32),
        pltpu.VMEM((2, 2, 128), jnp.int32),
        pltpu.VMEM((128,), jnp.float32),
        pltpu.VMEM((128,), jnp.float32),
        pltpu.VMEM((128,), jnp.float32),
        pltpu.VMEM((128,), jnp.float32),
        pltpu.VMEM((128,), jnp.float32),
        pltpu.VMEM((128,), jnp.float32),
        pltpu.VMEM((128, H), jnp.float32),
        pltpu.VMEM((128, H), jnp.float32),
        pltpu.VMEM((NPT,), jnp.float32),
        pltpu.VMEM((16,), jnp.float32),
        pltpu.VMEM_SHARED((NP,), jnp.float32),
        pltpu.VMEM_SHARED((NP, H), jnp.float32),
        pltpu.SemaphoreType.DMA,
        pltpu.SemaphoreType.DMA,
        pltpu.SemaphoreType.DMA,
        pltpu.SemaphoreType.DMA,
        pltpu.SemaphoreType.DMA,
        pltpu.SemaphoreType.DMA,
    ],
)


def _shift_const(mx):
  tmax = jnp.max(mx[:, 0, :64]) + jnp.max(mx[:, 0, 64:])
  return jnp.where(tmax > 0, tmax, 0.2 * tmax)


def kernel(x, edge_index, batch, W1, b1, Wg, att_src, att_dst, bg, W2, b2):
  f32 = jnp.float32
  xp = jnp.zeros((NP, D), f32).at[:N].set(x)
  loops = jnp.arange(N, dtype=jnp.int32)
  pad_src = (jnp.arange(PAD + 128, dtype=jnp.int32) * 37) % N
  pad_dst = N + (jnp.arange(PAD + 128, dtype=jnp.int32) % (NP - N))
  src_f = jnp.concatenate(
      [edge_index[0].astype(jnp.int32), loops, pad_src]).reshape(
          EROWS + 1, 1, 128)
  dst_f = jnp.concatenate(
      [edge_index[1].astype(jnp.int32), loops, pad_dst]).reshape(
          EROWS + 1, 1, 128)
  idx3 = jnp.concatenate([src_f, dst_f], axis=1)
  batchp = jnp.full((NP,), G, jnp.int32).at[:N].set(batch.astype(jnp.int32))

  h0, hw, asv, adv, mx = _tc1(xp, W1, b1.reshape(1, H), Wg[0],
                              att_src[0].reshape(H, 1),
                              att_dst[0].reshape(H, 1))
  h_in = h0
  for l in range(2):
    mvec = jnp.full((16,), _shift_const(mx), f32)
    o2, s2 = _sc_layer(idx3, asv.reshape(NP), adv.reshape(NP), mvec, hw)
    sb2 = s2.reshape(2, NP, 1)
    if l == 0:
      h_in, hw, asv, adv, mx = _tc2(o2[0], o2[1], sb2[0], sb2[1],
                                    bg[0].reshape(1, H), h_in, Wg[1],
                                    att_src[1].reshape(H, 1),
                                    att_dst[1].reshape(H, 1))
    else:
      _, out = _tc3(o2[0], o2[1], sb2[0], sb2[1], bg[1].reshape(1, H), h_in,
                    batchp, W2, b2.reshape(1, OUT))
  return out


# speedup vs baseline: 43.9816x; 1.0014x over previous
import jax
import jax.numpy as jnp
from jax import lax
from jax.experimental import pallas as pl
from jax.experimental.pallas import tpu as pltpu
from jax.experimental.pallas import tpu_sc as plsc

N = 10000
D = 128
H = 128
OUT = 128
G = 64
NP = 10240
NPT = NP // 16
EE = 320000 + N
NCHUNK = 82
EROWS = 32 * NCHUNK
EEP = EROWS * 128
PAD = EEP - EE
RB = 10
_HI = jax.lax.Precision.HIGHEST


def _dot(a, b):
  return jax.lax.dot(a, b, precision=_HI, preferred_element_type=jnp.float32)


def _tc1_body(x_ref, w1_ref, b1_ref, wg_ref, asrc_ref, adst_ref,
              h0_ref, hw_ref, asv_ref, adv_ref, mx_ref):
  h0 = _dot(x_ref[...], w1_ref[...]) + b1_ref[...]
  hw = _dot(h0, wg_ref[...])
  asv = _dot(hw, asrc_ref[...])
  adv = _dot(hw, adst_ref[...])
  h0_ref[...] = h0
  hw_ref[...] = hw
  asv_ref[...] = asv
  adv_ref[...] = adv
  ma = jnp.broadcast_to(jnp.max(asv), (1, 1, 64))
  mb = jnp.broadcast_to(jnp.max(adv), (1, 1, 64))
  mx_ref[...] = jnp.concatenate([ma, mb], axis=2)


_tc1 = pl.pallas_call(
    _tc1_body,
    grid=(RB,),
    in_specs=[
        pl.BlockSpec((1024, D), lambda i: (i, 0)),
        pl.BlockSpec((D, H), lambda i: (0, 0)),
        pl.BlockSpec((1, H), lambda i: (0, 0)),
        pl.BlockSpec((H, H), lambda i: (0, 0)),
        pl.BlockSpec((H, 1), lambda i: (0, 0)),
        pl.BlockSpec((H, 1), lambda i: (0, 0)),
    ],
    out_specs=[
        pl.BlockSpec((1024, H), lambda i: (i, 0)),
        pl.BlockSpec((1024, H), lambda i: (i, 0)),
        pl.BlockSpec((1024, 1), lambda i: (i, 0)),
        pl.BlockSpec((1024, 1), lambda i: (i, 0)),
        pl.BlockSpec((1, 1, 128), lambda i: (i, 0, 0)),
    ],
    out_shape=[
        jax.ShapeDtypeStruct((NP, H), jnp.float32),
        jax.ShapeDtypeStruct((NP, H), jnp.float32),
        jax.ShapeDtypeStruct((NP, 1), jnp.float32),
        jax.ShapeDtypeStruct((NP, 1), jnp.float32),
        jax.ShapeDtypeStruct((RB, 1, 128), jnp.float32),
    ],
)


def _tc2_body(o0_ref, o1_ref, sb0_ref, sb1_ref, bgl_ref, hin_ref, wg_ref,
              asrc_ref, adst_ref, h1_ref, hw_ref, asv_ref, adv_ref, mx_ref):
  denom = sb0_ref[...] + sb1_ref[...] + 1e-16
  agg = (o0_ref[...] + o1_ref[...]) / denom + bgl_ref[...]
  h1 = jnp.maximum(agg, 0.0) + hin_ref[...]
  hw = _dot(h1, wg_ref[...])
  asv = _dot(hw, asrc_ref[...])
  adv = _dot(hw, adst_ref[...])
  h1_ref[...] = h1
  hw_ref[...] = hw
  asv_ref[...] = asv
  adv_ref[...] = adv
  ma = jnp.broadcast_to(jnp.max(asv), (1, 1, 64))
  mb = jnp.broadcast_to(jnp.max(adv), (1, 1, 64))
  mx_ref[...] = jnp.concatenate([ma, mb], axis=2)


_tc2 = pl.pallas_call(
    _tc2_body,
    grid=(RB,),
    in_specs=[
        pl.BlockSpec((1024, H), lambda i: (i, 0)),
        pl.BlockSpec((1024, H), lambda i: (i, 0)),
        pl.BlockSpec((1024, 1), lambda i: (i, 0)),
        pl.BlockSpec((1024, 1), lambda i: (i, 0)),
        pl.BlockSpec((1, H), lambda i: (0, 0)),
        pl.BlockSpec((1024, H), lambda i: (i, 0)),
        pl.BlockSpec((H, H), lambda i: (0, 0)),
        pl.BlockSpec((H, 1), lambda i: (0, 0)),
        pl.BlockSpec((H, 1), lambda i: (0, 0)),
    ],
    out_specs=[
        pl.BlockSpec((1024, H), lambda i: (i, 0)),
        pl.BlockSpec((1024, H), lambda i: (i, 0)),
        pl.BlockSpec((1024, 1), lambda i: (i, 0)),
        pl.BlockSpec((1024, 1), lambda i: (i, 0)),
        pl.BlockSpec((1, 1, 128), lambda i: (i, 0, 0)),
    ],
    out_shape=[
        jax.ShapeDtypeStruct((NP, H), jnp.float32),
        jax.ShapeDtypeStruct((NP, H), jnp.float32),
        jax.ShapeDtypeStruct((NP, 1), jnp.float32),
        jax.ShapeDtypeStruct((NP, 1), jnp.float32),
        jax.ShapeDtypeStruct((RB, 1, 128), jnp.float32),
    ],
)


def _tc3_body(o0_ref, o1_ref, sb0_ref, sb1_ref, bgl_ref, hin_ref, batch_ref,
              w2_ref, b2_ref, pooled_ref, out_ref):
  i = pl.program_id(0)
  denom = sb0_ref[...] + sb1_ref[...] + 1e-16
  agg = (o0_ref[...] + o1_ref[...]) / denom + bgl_ref[...]
  h2 = jnp.maximum(agg, 0.0) + hin_ref[...]
  bt = batch_ref[...].reshape(1, 1024)
  gi = jax.lax.broadcasted_iota(jnp.int32, (G, 1024), 0)
  onehot = (gi == bt).astype(jnp.float32)
  part = _dot(onehot, h2)

  @pl.when(i == 0)
  def _():
    pooled_ref[...] = part

  @pl.when(i > 0)
  def _():
    pooled_ref[...] = pooled_ref[...] + part

  @pl.when(i == RB - 1)
  def _():
    out_ref[...] = _dot(pooled_ref[...], w2_ref[...]) + b2_ref[...]


_tc3 = pl.pallas_call(
    _tc3_body,
    grid=(RB,),
    in_specs=[
        pl.BlockSpec((1024, H), lambda i: (i, 0)),
        pl.BlockSpec((1024, H), lambda i: (i, 0)),
        pl.BlockSpec((1024, 1), lambda i: (i, 0)),
        pl.BlockSpec((1024, 1), lambda i: (i, 0)),
        pl.BlockSpec((1, H), lambda i: (0, 0)),
        pl.BlockSpec((1024, H), lambda i: (i, 0)),
        pl.BlockSpec((1024,), lambda i: (i,)),
        pl.BlockSpec((H, OUT), lambda i: (0, 0)),
        pl.BlockSpec((1, OUT), lambda i: (0, 0)),
    ],
    out_specs=[
        pl.BlockSpec((G, H), lambda i: (0, 0)),
        pl.BlockSpec((G, OUT), lambda i: (0, 0)),
    ],
    out_shape=[
        jax.ShapeDtypeStruct((G, H), jnp.float32),
        jax.ShapeDtypeStruct((G, OUT), jnp.float32),
    ],
)


def _sc_layer_body(idx3_h, asv_h, adv_h, mvec_h, hw_h,
                   o_out, sb_out,
                   idx_a, idxp, idxp2, asg_a, asg_b, adg_a, adg_b, w_a, w_b,
                   rows_a, rows_b, sv_t, m_t, s_sh, o_sh,
                   sg_a, sg_b, sr_a, sr_b, so_a, so_b, sem_i):
  cid = lax.axis_index("c")
  sid = lax.axis_index("s")
  z16 = jnp.zeros((16,), jnp.float32)
  tb = (cid * 16 + sid) * NCHUNK

  def prefetch(c, idx, asg, adg, rows, sg, sr):
    pltpu.sync_copy(idx3_h.at[c], idx)
    pltpu.async_copy(asv_h.at[idx.at[0]], asg, sg)
    pltpu.async_copy(adv_h.at[idx.at[1]], adg, sg)
    pltpu.async_copy(hw_h.at[idx.at[0]], rows, sr)

  def process(src_i, dst_i, asg, adg, w_t, rows, sg, sr, so):
    pltpu.make_async_copy(asv_h.at[src_i], asg, sg).wait()
    pltpu.make_async_copy(adv_h.at[dst_i], adg, sg).wait()
    m16 = m_t[...]
    for i in range(8):
      sl = pl.ds(16 * i, 16)
      t = asg[sl] + adg[sl]
      e = jnp.where(t > 0, t, 0.2 * t)
      w_t[sl] = jnp.exp(e - m16)
    pltpu.make_async_copy(hw_h.at[src_i], rows, sr).wait()

    @pl.loop(0, 8)
    def _scale(g):
      v16 = w_t[pl.ds(g * 16, 16)]
      for j in range(16):
        wb = jnp.full((16,), v16[j], jnp.float32)
        base = g * 16 + j
        for k in range(8):
          sl = pl.ds(16 * k, 16)
          rows[base, sl] = rows[base, sl] * wb

    pltpu.async_copy(w_t, s_sh.at[dst_i], so, add=True)
    pltpu.async_copy(rows, o_sh.at[dst_i], so, add=True)

  def drain(dst_i, w_t, rows, so):
    pltpu.make_async_copy(w_t, s_sh.at[dst_i], so).wait()
    pltpu.make_async_copy(rows, o_sh.at[dst_i], so).wait()

  @pl.loop(0, 128, unroll=4)
  def _zrows(j):
    for k in range(8):
      rows_b[j, pl.ds(16 * k, 16)] = z16

  for j in range(NPT // 16):
    sv_t[pl.ds(16 * j, 16)] = z16
  pltpu.sync_copy(mvec_h, m_t)
  prefetch(tb, idx_a, asg_a, adg_a, rows_a, sg_a, sr_a)
  pltpu.sync_copy(idx3_h.at[pl.ds(tb + 1, 2)], idxp)
  pltpu.async_copy(idx3_h.at[pl.ds(tb + 3, 2)], idxp2, sem_i)
  for b in range(NPT // 128):
    pltpu.sync_copy(rows_b, o_sh.at[pl.ds(sid * NPT + b * 128, 128)])
  pltpu.sync_copy(sv_t, s_sh.at[pl.ds(sid * NPT, NPT)])
  plsc.subcore_barrier()

  @pl.loop(0, NCHUNK // 2)
  def _pair(t):
    c0 = tb + 2 * t

    @pl.when(t > 0)
    def _():
      drain(idxp.at[0, 1], w_b, rows_b, so_b)
      pltpu.make_async_copy(idx3_h.at[pl.ds(c0 + 1, 2)], idxp2, sem_i).wait()
      for r2 in range(2):
        for r in range(2):
          for i in range(8):
            sl = pl.ds(16 * i, 16)
            idxp[r2, r, sl] = idxp2[r2, r, sl]

    @pl.when((t > 0) & (t < NCHUNK // 2 - 1))
    def _():
      pltpu.async_copy(idx3_h.at[pl.ds(c0 + 3, 2)], idxp2, sem_i)

    pltpu.async_copy(asv_h.at[idxp.at[0, 0]], asg_b, sg_b)
    pltpu.async_copy(adv_h.at[idxp.at[0, 1]], adg_b, sg_b)
    pltpu.async_copy(hw_h.at[idxp.at[0, 0]], rows_b, sr_b)
    process(idx_a.at[0], idx_a.at[1], asg_a, adg_a, w_a, rows_a,
            sg_a, sr_a, so_a)
    process(idxp.at[0, 0], idxp.at[0, 1], asg_b, adg_b, w_b, rows_b,
            sg_b, sr_b, so_b)

    @pl.when(t < NCHUNK // 2 - 1)
    def _():
      drain(idx_a.at[1], w_a, rows_a, so_a)
      for r in range(2):
        for i in range(8):
          sl = pl.ds(16 * i, 16)
          idx_a[r, sl] = idxp[1, r, sl]
      pltpu.async_copy(asv_h.at[idx_a.at[0]], asg_a, sg_a)
      pltpu.async_copy(adv_h.at[idx_a.at[1]], adg_a, sg_a)
      pltpu.async_copy(hw_h.at[idx_a.at[0]], rows_a, sr_a)

  drain(idx_a.at[1], w_a, rows_a, so_a)
  drain(idxp.at[0, 1], w_b, rows_b, so_b)
  plsc.subcore_barrier()

  pltpu.sync_copy(o_sh.at[pl.ds(sid * NPT, NPT)],
                  o_out.at[cid, pl.ds(sid * NPT, NPT)])
  pltpu.sync_copy(s_sh.at[pl.ds(sid * NPT, NPT)],
                  sb_out.at[pl.ds(cid * NP + sid * NPT, NPT)])


_sc_layer = pl.kernel(
    _sc_layer_body,
    out_type=(
        jax.ShapeDtypeStruct((2, NP, H), jnp.float32),
        jax.ShapeDtypeStruct((2 * NP,), jnp.float32),
    ),
    mesh=plsc.VectorSubcoreMesh(core_axis_name="c", subcore_axis_name="s",
                                num_cores=2, num_subcores=16),
    compiler_params=pltpu.CompilerParams(needs_layout_passes=False),
    scratch_types=[
        pltpu.VMEM((2, 128), jnp.int32),
        pltpu.VMEM((2, 2, 128), jnp.int32),
        pltpu.VMEM((2, 2, 128), jnp.int32),
        pltpu.VMEM((128,), jnp.float32),
        pltpu.VMEM((128,), jnp.float32),
        pltpu.VMEM((128,), jnp.float32),
        pltpu.VMEM((128,), jnp.float32),
        pltpu.VMEM((128,), jnp.float32),
        pltpu.VMEM((128,), jnp.float32),
        pltpu.VMEM((128, H), jnp.float32),
        pltpu.VMEM((128, H), jnp.float32),
        pltpu.VMEM((NPT,), jnp.float32),
        pltpu.VMEM((16,), jnp.float32),
        pltpu.VMEM_SHARED((NP,), jnp.float32),
        pltpu.VMEM_SHARED((NP, H), jnp.float32),
        pltpu.SemaphoreType.DMA,
        pltpu.SemaphoreType.DMA,
        pltpu.SemaphoreType.DMA,
        pltpu.SemaphoreType.DMA,
        pltpu.SemaphoreType.DMA,
        pltpu.SemaphoreType.DMA,
        pltpu.SemaphoreType.DMA,
    ],
)


def _shift_const(mx):
  tmax = jnp.max(mx[:, 0, :64]) + jnp.max(mx[:, 0, 64:])
  return jnp.where(tmax > 0, tmax, 0.2 * tmax)


def kernel(x, edge_index, batch, W1, b1, Wg, att_src, att_dst, bg, W2, b2):
  f32 = jnp.float32
  xp = jnp.zeros((NP, D), f32).at[:N].set(x)
  loops = jnp.arange(N, dtype=jnp.int32)
  pad_src = (jnp.arange(PAD + 128, dtype=jnp.int32) * 37) % N
  pad_dst = N + (jnp.arange(PAD + 128, dtype=jnp.int32) % (NP - N))
  src_f = jnp.concatenate(
      [edge_index[0].astype(jnp.int32), loops, pad_src]).reshape(
          EROWS + 1, 1, 128)
  dst_f = jnp.concatenate(
      [edge_index[1].astype(jnp.int32), loops, pad_dst]).reshape(
          EROWS + 1, 1, 128)
  idx3 = jnp.concatenate([src_f, dst_f], axis=1)
  batchp = jnp.full((NP,), G, jnp.int32).at[:N].set(batch.astype(jnp.int32))

  h0, hw, asv, adv, mx = _tc1(xp, W1, b1.reshape(1, H), Wg[0],
                              att_src[0].reshape(H, 1),
                              att_dst[0].reshape(H, 1))
  h_in = h0
  for l in range(2):
    mvec = jnp.full((16,), _shift_const(mx), f32)
    o2, s2 = _sc_layer(idx3, asv.reshape(NP), adv.reshape(NP), mvec, hw)
    sb2 = s2.reshape(2, NP, 1)
    if l == 0:
      h_in, hw, asv, adv, mx = _tc2(o2[0], o2[1], sb2[0], sb2[1],
                                    bg[0].reshape(1, H), h_in, Wg[1],
                                    att_src[1].reshape(H, 1),
                                    att_dst[1].reshape(H, 1))
    else:
      _, out = _tc3(o2[0], o2[1], sb2[0], sb2[1], bg[1].reshape(1, H), h_in,
                    batchp, W2, b2.reshape(1, OUT))
  return out
